# baseline (speedup 1.0000x reference)
import jax
import jax.numpy as jnp
import numpy as np
from jax.experimental import pallas as pl
from jax.experimental.pallas import tpu as pltpu

_B, _N_C, _N_T = 16, 512, 128
_K = 8
_H = 64
_NUM_BLKS = 6
_N_ALL = _N_C + _N_T
_INV_SQRT_H = 1.0 / np.sqrt(_H)
_NEG = -1e30


def _layernorm(x, g, b, eps=1e-6):
    mu = jnp.mean(x, axis=-1, keepdims=True)
    xc = x - mu
    var = jnp.mean(xc * xc, axis=-1, keepdims=True)
    return xc * jax.lax.rsqrt(var + eps) * g + b


def _mm(a, b):
    return jnp.dot(a, b, preferred_element_type=jnp.float32)


def _sgnp_kernel(
    x_in_ref,
    s_all_ref,
    s_ctx_t_ref,
    eW1, eb1, eW2, eb2, eW3, eb3,
    ng, nb,
    Wqkv,
    We_cat,
    Wo,
    ln1g, ln1b, ffW1, ffb1, ffW2, ffb2, ln2g, ln2b,
    hW1, hb1, hW2, hb2, hW3, hb3,
    out_ref,
):
    x = x_in_ref[0]

    x = jax.nn.gelu(_mm(x, eW1[...]) + eb1[...])
    x = jax.nn.gelu(_mm(x, eW2[...]) + eb2[...])
    x = _mm(x, eW3[...]) + eb3[...]
    h = _layernorm(x, ng[...], nb[...])

    s_all = s_all_ref[0]
    kx = s_ctx_t_ref[0]
    d2 = (s_all[:, 0:1] - kx[0:1, :]) ** 2 + (s_all[:, 1:2] - kx[1:2, :]) ** 2

    col = jax.lax.broadcasted_iota(jnp.int32, (_N_ALL, _N_C), 1)
    neg_add = jnp.full((_N_ALL, _N_C), _NEG, jnp.float32)
    for _ in range(_K):
        m = jnp.min(d2, axis=1, keepdims=True)
        cand = jnp.where(d2 == m, col, _N_C)
        cidx = jnp.min(cand, axis=1, keepdims=True)
        onehot = col == cidx
        neg_add = jnp.where(onehot, 0.0, neg_add)
        d2 = jnp.where(onehot, jnp.float32(np.inf), d2)

    e_cat = _mm(s_all, We_cat[...])

    for blk in range(_NUM_BLKS):
        e_all = e_cat[:, blk * _H:(blk + 1) * _H]
        e_ctx = e_all[:_N_C, :]
        hqkv = _mm(h, Wqkv[blk])
        q = hqkv[:, :_H] * _INV_SQRT_H
        ke = hqkv[:_N_C, _H:2 * _H] + e_ctx
        ve = hqkv[:_N_C, 2 * _H:] + e_ctx
        s_mat = jax.lax.dot_general(
            q, ke, (((1,), (1,)), ((), ())),
            preferred_element_type=jnp.float32)
        logits = s_mat + neg_add
        mx = jnp.max(logits, axis=1, keepdims=True)
        ex = jnp.exp(logits - mx)
        den = jnp.sum(ex, axis=1, keepdims=True)
        recip = 1.0 / (den + 1e-9)
        msg = _mm(ex, ve) * recip - (den * recip) * e_all
        h = _layernorm(h + _mm(msg, Wo[blk]), ln1g[blk], ln1b[blk])
        ff = _mm(jax.nn.gelu(_mm(h, ffW1[blk]) + ffb1[blk]), ffW2[blk]) + ffb2[blk]
        h = _layernorm(h + ff, ln2g[blk], ln2b[blk])

    xt = h[_N_C:, :]
    xt = jax.nn.gelu(_mm(xt, hW1[...]) + hb1[...])
    xt = jax.nn.gelu(_mm(xt, hW2[...]) + hb2[...])
    f = _mm(xt, hW3[...]) + hb3[...]
    mean = f[:, 0:1]
    scale = jax.nn.softplus(f[:, 1:2])
    out_ref[0] = jnp.concatenate([mean, scale], axis=1)


def kernel(s_ctx, f_ctx, s_test, params):
    p = params
    f32 = jnp.float32

    e_obs = jnp.broadcast_to(p['embed_obs'][1], (_B, _N_C, 4))
    e_unobs = jnp.broadcast_to(p['embed_obs'][0], (_B, _N_T, 4))
    ctx_in = jnp.concatenate([e_obs, s_ctx, f_ctx], axis=-1)
    test_in = jnp.concatenate(
        [e_unobs, s_test, jnp.zeros((_B, _N_T, 1), f32)], axis=-1)
    x_in = jnp.concatenate([ctx_in, test_in], axis=1)
    x_in = jnp.pad(x_in, ((0, 0), (0, 0), (0, 1)))
    s_all = jnp.concatenate([s_ctx, s_test], axis=1)
    s_ctx_t = jnp.swapaxes(s_ctx, 1, 2)

    eW1 = jnp.pad(p['embed_all_W'][0], ((0, 1), (0, 0)))
    eW2, eW3 = p['embed_all_W'][1], p['embed_all_W'][2]
    eb1, eb2, eb3 = (b.reshape(1, -1) for b in p['embed_all_b'])
    ng = p['norm_g'].reshape(1, _H)
    nb = p['norm_b'].reshape(1, _H)
    r2 = lambda a: a.reshape(_NUM_BLKS, 1, -1)
    hb1, hb2, hb3 = (b.reshape(1, -1) for b in p['head_b'])

    Wqkv = jnp.concatenate([p['gat_Wq'], p['gat_Wk'], p['gat_Wv']], axis=2)
    We_cat = jnp.transpose(p['gat_We'], (1, 0, 2)).reshape(2, _NUM_BLKS * _H)

    ops = [
        x_in, s_all, s_ctx_t,
        eW1, eb1, eW2, eb2, eW3, eb3, ng, nb,
        Wqkv, We_cat, p['gat_Wo'],
        r2(p['gat_ln1_g']), r2(p['gat_ln1_b']),
        p['gat_ffn_W1'], r2(p['gat_ffn_b1']),
        p['gat_ffn_W2'], r2(p['gat_ffn_b2']),
        r2(p['gat_ln2_g']), r2(p['gat_ln2_b']),
        p['head_W'][0], hb1, p['head_W'][1], hb2, p['head_W'][2], hb3,
    ]

    def whole(a):
        return pl.BlockSpec(a.shape, lambda b: (0,) * a.ndim)

    in_specs = [
        pl.BlockSpec((1, _N_ALL, 8), lambda b: (b, 0, 0)),
        pl.BlockSpec((1, _N_ALL, 2), lambda b: (b, 0, 0)),
        pl.BlockSpec((1, 2, _N_C), lambda b: (b, 0, 0)),
    ] + [whole(a) for a in ops[3:]]

    out = pl.pallas_call(
        _sgnp_kernel,
        grid=(_B,),
        in_specs=in_specs,
        out_specs=pl.BlockSpec((1, _N_T, 2), lambda b: (b, 0, 0)),
        out_shape=jax.ShapeDtypeStruct((_B, _N_T, 2), f32),
        compiler_params=pltpu.CompilerParams(
            dimension_semantics=("arbitrary",),
        ),
    )(*ops)
    return out


# speedup vs baseline: 1.5365x; 1.5365x over previous
import jax
import jax.numpy as jnp
import numpy as np
from jax.experimental import pallas as pl
from jax.experimental.pallas import tpu as pltpu

_B, _N_C, _N_T = 16, 512, 128
_K = 8
_H = 64
_NUM_BLKS = 6
_N_ALL = _N_C + _N_T
_INV_SQRT_H = 1.0 / np.sqrt(_H)
_NEG = -1e30


def _layernorm(x, g, b, eps=1e-6):
    mu = jnp.mean(x, axis=-1, keepdims=True)
    xc = x - mu
    var = jnp.mean(xc * xc, axis=-1, keepdims=True)
    return xc * jax.lax.rsqrt(var + eps) * g + b


def _mm(a, b):
    return jnp.dot(a, b, preferred_element_type=jnp.float32)


_PB = 2


def _sgnp_kernel(
    x_in_ref,
    s_all_ref,
    s_ctx_t_ref,
    eW1, eb1, eW2, eb2, eW3, eb3,
    ng, nb,
    Wq, Wk, Wv, We, Wo,
    ln1g, ln1b, ffW1, ffb1, ffW2, ffb2, ln2g, ln2b,
    hW1, hb1, hW2, hb2, hW3, hb3,
    out_ref,
):
    n_rows = _PB * _N_ALL
    x = x_in_ref[...].reshape(n_rows, 8)

    x = jax.nn.gelu(_mm(x, eW1[...]) + eb1[...])
    x = jax.nn.gelu(_mm(x, eW2[...]) + eb2[...])
    x = _mm(x, eW3[...]) + eb3[...]
    h = _layernorm(x, ng[...], nb[...])

    s_all = s_all_ref[...]
    d2s = []
    for p in range(_PB):
        kx = s_ctx_t_ref[p]
        sq = s_all[p]
        d2s.append((sq[:, 0:1] - kx[0:1, :]) ** 2 + (sq[:, 1:2] - kx[1:2, :]) ** 2)
    d2 = jnp.concatenate(d2s, axis=0)

    col = jax.lax.broadcasted_iota(jnp.int32, (n_rows, _N_C), 1).astype(jnp.float32)
    neg_add = jnp.full((n_rows, _N_C), _NEG, jnp.float32)
    for _ in range(_K):
        m = jnp.min(d2, axis=1, keepdims=True)
        cand = jnp.where(d2 == m, col, jnp.float32(_N_C))
        cidx = jnp.min(cand, axis=1, keepdims=True)
        onehot = col == cidx
        neg_add = jnp.where(onehot, 0.0, neg_add)
        d2 = jnp.where(onehot, jnp.float32(np.inf), d2)

    for blk in range(_NUM_BLKS):
        q = _mm(h, Wq[blk]) * _INV_SQRT_H
        s_parts = []
        ves = []
        e_alls = []
        for p in range(_PB):
            hp = h[p * _N_ALL:p * _N_ALL + _N_C, :]
            e_all = _mm(s_all[p], We[blk])
            e_ctx = e_all[:_N_C, :]
            ke = _mm(hp, Wk[blk]) + e_ctx
            ves.append(_mm(hp, Wv[blk]) + e_ctx)
            e_alls.append(e_all)
            s_parts.append(jax.lax.dot_general(
                q[p * _N_ALL:(p + 1) * _N_ALL, :], ke,
                (((1,), (1,)), ((), ())),
                preferred_element_type=jnp.float32))
        s_mat = jnp.concatenate(s_parts, axis=0)
        logits = s_mat + neg_add
        mx = jnp.max(logits, axis=1, keepdims=True)
        ex = jnp.exp(logits - mx)
        den = jnp.sum(ex, axis=1, keepdims=True)
        recip = 1.0 / (den + 1e-9)
        raw = jnp.concatenate(
            [_mm(ex[p * _N_ALL:(p + 1) * _N_ALL, :], ves[p]) for p in range(_PB)],
            axis=0)
        e_cat = jnp.concatenate(e_alls, axis=0)
        msg = raw * recip - (den * recip) * e_cat
        h = _layernorm(h + _mm(msg, Wo[blk]), ln1g[blk], ln1b[blk])
        ff = _mm(jax.nn.gelu(_mm(h, ffW1[blk]) + ffb1[blk]), ffW2[blk]) + ffb2[blk]
        h = _layernorm(h + ff, ln2g[blk], ln2b[blk])

    xt = jnp.concatenate(
        [h[p * _N_ALL + _N_C:(p + 1) * _N_ALL, :] for p in range(_PB)], axis=0)
    xt = jax.nn.gelu(_mm(xt, hW1[...]) + hb1[...])
    xt = jax.nn.gelu(_mm(xt, hW2[...]) + hb2[...])
    f = _mm(xt, hW3[...]) + hb3[...]
    mean = f[:, 0:1]
    scale = jax.nn.softplus(f[:, 1:2])
    out_ref[...] = jnp.concatenate([mean, scale], axis=1).reshape(_PB, _N_T, 2)


def kernel(s_ctx, f_ctx, s_test, params):
    p = params
    f32 = jnp.float32

    e_obs = jnp.broadcast_to(p['embed_obs'][1], (_B, _N_C, 4))
    e_unobs = jnp.broadcast_to(p['embed_obs'][0], (_B, _N_T, 4))
    ctx_in = jnp.concatenate([e_obs, s_ctx, f_ctx], axis=-1)
    test_in = jnp.concatenate(
        [e_unobs, s_test, jnp.zeros((_B, _N_T, 1), f32)], axis=-1)
    x_in = jnp.concatenate([ctx_in, test_in], axis=1)
    x_in = jnp.pad(x_in, ((0, 0), (0, 0), (0, 1)))
    s_all = jnp.concatenate([s_ctx, s_test], axis=1)
    s_ctx_t = jnp.swapaxes(s_ctx, 1, 2)

    eW1 = jnp.pad(p['embed_all_W'][0], ((0, 1), (0, 0)))
    eW2, eW3 = p['embed_all_W'][1], p['embed_all_W'][2]
    eb1, eb2, eb3 = (b.reshape(1, -1) for b in p['embed_all_b'])
    ng = p['norm_g'].reshape(1, _H)
    nb = p['norm_b'].reshape(1, _H)
    r2 = lambda a: a.reshape(_NUM_BLKS, 1, -1)
    hb1, hb2, hb3 = (b.reshape(1, -1) for b in p['head_b'])

    ops = [
        x_in, s_all, s_ctx_t,
        eW1, eb1, eW2, eb2, eW3, eb3, ng, nb,
        p['gat_Wq'], p['gat_Wk'], p['gat_Wv'], p['gat_We'], p['gat_Wo'],
        r2(p['gat_ln1_g']), r2(p['gat_ln1_b']),
        p['gat_ffn_W1'], r2(p['gat_ffn_b1']),
        p['gat_ffn_W2'], r2(p['gat_ffn_b2']),
        r2(p['gat_ln2_g']), r2(p['gat_ln2_b']),
        p['head_W'][0], hb1, p['head_W'][1], hb2, p['head_W'][2], hb3,
    ]

    def whole(a):
        return pl.BlockSpec(a.shape, lambda b: (0,) * a.ndim)

    in_specs = [
        pl.BlockSpec((_PB, _N_ALL, 8), lambda b: (b, 0, 0)),
        pl.BlockSpec((_PB, _N_ALL, 2), lambda b: (b, 0, 0)),
        pl.BlockSpec((_PB, 2, _N_C), lambda b: (b, 0, 0)),
    ] + [whole(a) for a in ops[3:]]

    out = pl.pallas_call(
        _sgnp_kernel,
        grid=(_B // _PB,),
        in_specs=in_specs,
        out_specs=pl.BlockSpec((_PB, _N_T, 2), lambda b: (b, 0, 0)),
        out_shape=jax.ShapeDtypeStruct((_B, _N_T, 2), f32),
        compiler_params=pltpu.CompilerParams(
            dimension_semantics=("arbitrary",),
        ),
    )(*ops)
    return out
